# Initial kernel scaffold; baseline (speedup 1.0000x reference)
import jax
import jax.numpy as jnp
from jax.experimental import pallas as pl


def kernel(xs, coordinates):
    raise NotImplementedError("write your pallas kernel here")


# speedup vs baseline: 6.0524x; 6.0524x over previous
import functools

import jax
import jax.numpy as jnp
from jax import lax
from jax.experimental import pallas as pl
from jax.experimental.pallas import tpu as pltpu
from jax.experimental.pallas import tpu_sc as plsc

N = 8192
D = 64
B = 256
K = 8
R = 256
M = N * K

NC = 2
NS = 16
L = 16
NW = NC * NS
XS_PER = B // NW
CO_PER = D // NW
CHUNK = 2048


def _topk_body(cb_ref, call_ref, out_ref):
    cb = cb_ref[...]
    call = call_ref[...]
    dot = lax.dot_general(cb, call, (((0,), (0,)), ((), ())),
                          preferred_element_type=jnp.float32)
    xx = jnp.sum(call * call, axis=0)[None, :]
    yy = jnp.sum(cb * cb, axis=0)[:, None]
    d = jnp.maximum(xx + yy - 2.0 * dot, 0.0)
    iota = lax.broadcasted_iota(jnp.int32, (R, N), 1)
    cols = []
    for _ in range(K):
        m = jnp.min(d, axis=1, keepdims=True)
        idxn = jnp.min(jnp.where(d == m, iota, N), axis=1, keepdims=True)
        cols.append(idxn)
        d = jnp.where(iota == idxn, jnp.float32(jnp.inf), d)
    out_ref[...] = jnp.concatenate(cols, axis=1)


def _neighbor_indices(coordinates):
    return pl.pallas_call(
        _topk_body,
        grid=(N // R,),
        in_specs=[
            pl.BlockSpec((D, R), lambda i: (0, i)),
            pl.BlockSpec((D, N), lambda i: (0, 0)),
        ],
        out_specs=pl.BlockSpec((R, K), lambda i: (i, 0)),
        out_shape=jax.ShapeDtypeStruct((N, K), jnp.int32),
    )(coordinates, coordinates)


def _gather_body(xs_hbm, co_hbm, flat_hbm, oxs_hbm, oco_hbm,
                 xsrows, corows, idxbuf, obx, obc):
    c = lax.axis_index("c")
    s = lax.axis_index("s")
    wid = s * NC + c
    pltpu.sync_copy(xs_hbm.at[pl.ds(wid * XS_PER, XS_PER)], xsrows)
    pltpu.sync_copy(co_hbm.at[pl.ds(wid * CO_PER, CO_PER)], corows)

    def chunk_body(ci, carry):
        base = ci * CHUNK
        pltpu.sync_copy(flat_hbm.at[pl.ds(base, CHUNK)], idxbuf)

        def step(j, carry2):
            iv = idxbuf[pl.ds(j * L, L)]
            for r in range(XS_PER):
                rv = jnp.full((L,), r, dtype=jnp.int32)
                obx[r, pl.ds(j * L, L)] = plsc.load_gather(xsrows, [rv, iv])
            for r in range(CO_PER):
                rv = jnp.full((L,), r, dtype=jnp.int32)
                obc[r, pl.ds(j * L, L)] = plsc.load_gather(corows, [rv, iv])
            return carry2

        lax.fori_loop(0, CHUNK // L, step, 0, unroll=2)
        for r in range(XS_PER):
            pltpu.sync_copy(obx.at[r],
                            oxs_hbm.at[wid * XS_PER + r, pl.ds(base, CHUNK)])
        for r in range(CO_PER):
            pltpu.sync_copy(obc.at[r],
                            oco_hbm.at[wid * CO_PER + r, pl.ds(base, CHUNK)])
        return carry

    lax.fori_loop(0, M // CHUNK, chunk_body, 0)


@functools.lru_cache(maxsize=1)
def _gather_kernel():
    return pl.kernel(
        _gather_body,
        out_type=(
            jax.ShapeDtypeStruct((B, M), jnp.float32),
            jax.ShapeDtypeStruct((D, M), jnp.float32),
        ),
        mesh=plsc.VectorSubcoreMesh(
            core_axis_name="c", subcore_axis_name="s",
            num_cores=NC, num_subcores=NS,
        ),
        compiler_params=pltpu.CompilerParams(
            use_tc_tiling_on_sc=False, needs_layout_passes=False),
        scratch_types=[
            pltpu.VMEM((XS_PER, N), jnp.float32),
            pltpu.VMEM((CO_PER, N), jnp.float32),
            pltpu.VMEM((CHUNK,), jnp.int32),
            pltpu.VMEM((XS_PER, CHUNK), jnp.float32),
            pltpu.VMEM((CO_PER, CHUNK), jnp.float32),
        ],
    )


def kernel(xs, coordinates):
    idx = _neighbor_indices(coordinates)
    flat = idx.reshape(-1).at[0].set(0)
    oxs, oco = _gather_kernel()(xs, coordinates, flat)
    return oxs[:, None, :, None], oco[:, None, :, None]


# speedup vs baseline: 6.6762x; 1.1031x over previous
import functools

import jax
import jax.numpy as jnp
from jax import lax
from jax.experimental import pallas as pl
from jax.experimental.pallas import tpu as pltpu
from jax.experimental.pallas import tpu_sc as plsc

N = 8192
D = 64
B = 256
K = 8
R = 256
M = N * K

NC = 2
NS = 16
L = 16
NW = NC * NS
XS_PER = B // NW
CO_PER = D // NW
CHUNK = 2048


def _topk_body(cb_ref, call_ref, out_ref):
    cb = cb_ref[...]
    call = call_ref[...]
    dot = lax.dot_general(cb, call, (((0,), (0,)), ((), ())),
                          preferred_element_type=jnp.float32)
    xx = jnp.sum(call * call, axis=0)[None, :]
    yy = jnp.sum(cb * cb, axis=0)[:, None]
    d = jnp.maximum(xx + yy - 2.0 * dot, 0.0)
    iota = lax.broadcasted_iota(jnp.int32, (R, N), 1)
    cols = []
    for _ in range(K):
        m = jnp.min(d, axis=1, keepdims=True)
        idxn = jnp.min(jnp.where(d == m, iota, N), axis=1, keepdims=True)
        cols.append(idxn)
        d = jnp.where(iota == idxn, jnp.float32(jnp.inf), d)
    out_ref[...] = jnp.concatenate(cols, axis=1)


def _neighbor_indices(coordinates):
    return pl.pallas_call(
        _topk_body,
        grid=(N // R,),
        in_specs=[
            pl.BlockSpec((D, R), lambda i: (0, i)),
            pl.BlockSpec((D, N), lambda i: (0, 0)),
        ],
        out_specs=pl.BlockSpec((R, K), lambda i: (i, 0)),
        out_shape=jax.ShapeDtypeStruct((N, K), jnp.int32),
    )(coordinates, coordinates)


NCH = M // CHUNK


def _gather_body(xs_hbm, co_hbm, flat_hbm, oxs_hbm, oco_hbm,
                 xsrows, corows, idxbuf, obx, obc, isem, osem):
    c = lax.axis_index("c")
    s = lax.axis_index("s")
    wid = s * NC + c
    pltpu.sync_copy(xs_hbm.at[pl.ds(wid * XS_PER, XS_PER)], xsrows)
    pltpu.sync_copy(co_hbm.at[pl.ds(wid * CO_PER, CO_PER)], corows)

    def idx_start(ci, b):
        pltpu.async_copy(flat_hbm.at[pl.ds(ci * CHUNK, CHUNK)],
                         idxbuf.at[b], isem.at[b])

    def idx_wait(b):
        pltpu.make_async_copy(flat_hbm.at[pl.ds(0, CHUNK)],
                              idxbuf.at[b], isem.at[b]).wait()

    def gather_chunk(b):
        def step(j, carry):
            iv = idxbuf[b, pl.ds(j * L, L)]
            for r in range(XS_PER):
                rv = jnp.full((L,), r, dtype=jnp.int32)
                obx[b, r, pl.ds(j * L, L)] = plsc.load_gather(xsrows, [rv, iv])
            for r in range(CO_PER):
                rv = jnp.full((L,), r, dtype=jnp.int32)
                obc[b, r, pl.ds(j * L, L)] = plsc.load_gather(corows, [rv, iv])
            return carry

        lax.fori_loop(0, CHUNK // L, step, 0, unroll=4)

    def out_start(ci, b):
        base = ci * CHUNK
        for r in range(XS_PER):
            pltpu.async_copy(obx.at[b, r],
                             oxs_hbm.at[wid * XS_PER + r, pl.ds(base, CHUNK)],
                             osem.at[b])
        for r in range(CO_PER):
            pltpu.async_copy(obc.at[b, r],
                             oco_hbm.at[wid * CO_PER + r, pl.ds(base, CHUNK)],
                             osem.at[b])

    def out_wait(b):
        for r in range(XS_PER):
            pltpu.make_async_copy(obx.at[b, r],
                                  oxs_hbm.at[wid * XS_PER + r, pl.ds(0, CHUNK)],
                                  osem.at[b]).wait()
        for r in range(CO_PER):
            pltpu.make_async_copy(obc.at[b, r],
                                  oco_hbm.at[wid * CO_PER + r, pl.ds(0, CHUNK)],
                                  osem.at[b]).wait()

    idx_start(0, 0)
    idx_start(1, 1)
    idx_wait(0)
    gather_chunk(0)
    idx_start(2, 0)
    out_start(0, 0)
    idx_wait(1)
    gather_chunk(1)
    idx_start(3, 1)
    out_start(1, 1)

    def pair(g, carry):
        ci0 = g * 2
        out_wait(0)
        idx_wait(0)
        gather_chunk(0)
        idx_start(ci0 + 2, 0)
        out_start(ci0, 0)
        out_wait(1)
        idx_wait(1)
        gather_chunk(1)
        idx_start(ci0 + 3, 1)
        out_start(ci0 + 1, 1)
        return carry

    lax.fori_loop(1, NCH // 2 - 1, pair, 0)

    out_wait(0)
    idx_wait(0)
    gather_chunk(0)
    out_start(NCH - 2, 0)
    out_wait(1)
    idx_wait(1)
    gather_chunk(1)
    out_start(NCH - 1, 1)
    out_wait(0)
    out_wait(1)


@functools.lru_cache(maxsize=1)
def _gather_kernel():
    return pl.kernel(
        _gather_body,
        out_type=(
            jax.ShapeDtypeStruct((B, M), jnp.float32),
            jax.ShapeDtypeStruct((D, M), jnp.float32),
        ),
        mesh=plsc.VectorSubcoreMesh(
            core_axis_name="c", subcore_axis_name="s",
            num_cores=NC, num_subcores=NS,
        ),
        compiler_params=pltpu.CompilerParams(
            use_tc_tiling_on_sc=False, needs_layout_passes=False),
        scratch_types=[
            pltpu.VMEM((XS_PER, N), jnp.float32),
            pltpu.VMEM((CO_PER, N), jnp.float32),
            pltpu.VMEM((2, CHUNK), jnp.int32),
            pltpu.VMEM((2, XS_PER, CHUNK), jnp.float32),
            pltpu.VMEM((2, CO_PER, CHUNK), jnp.float32),
            pltpu.SemaphoreType.DMA((2,)),
            pltpu.SemaphoreType.DMA((2,)),
        ],
    )


def kernel(xs, coordinates):
    idx = _neighbor_indices(coordinates)
    flat = idx.reshape(-1).at[0].set(0)
    oxs, oco = _gather_kernel()(xs, coordinates, flat)
    return oxs[:, None, :, None], oco[:, None, :, None]
